# baseline (device time: 197888 ns/iter reference)
import jax
import jax.numpy as jnp
from jax import lax
from jax.experimental import pallas as pl
from jax.experimental.pallas import tpu as pltpu

B, S, HD_IN, HD_OUT = 4, 1024, 2048, 4096
S_HALF = S // 2
C = HD_OUT // 2
NC = 8
R = B * S_HALF // NC


def kernel(O, Wo):
    H, D = O.shape[2], O.shape[3]
    Wo2 = Wo.astype(jnp.bfloat16)

    def body(
        o_hbm, wo_ref, out_ref,
        land_rem, land_own, xsend, xrecv, yrecv, own,
        xsend_sems, xrecv_sems, fsend_sems, yrecv_sems,
        rem_sem, own_sem,
    ):
        my_x = lax.axis_index("x")
        my_y = lax.axis_index("y")
        x_nbr = (1 - my_x, my_y)
        y_nbr = (my_x, 1 - my_y)

        barrier_sem = pltpu.get_barrier_semaphore()
        for nbr in (x_nbr, y_nbr):
            pl.semaphore_signal(
                barrier_sem, inc=1,
                device_id=nbr, device_id_type=pl.DeviceIdType.MESH,
            )
        pl.semaphore_wait(barrier_sem, 2)

        own_rows = my_x * S_HALF
        rem_rows = (1 - my_x) * S_HALF

        def xrdma(c):
            return pltpu.make_async_remote_copy(
                src_ref=xsend.at[c % 2],
                dst_ref=xrecv.at[c],
                send_sem=xsend_sems.at[c % 2],
                recv_sem=xrecv_sems.at[c],
                device_id=x_nbr,
                device_id_type=pl.DeviceIdType.MESH,
            )

        def fwd(c):
            return pltpu.make_async_remote_copy(
                src_ref=xrecv.at[c],
                dst_ref=yrecv.at[c],
                send_sem=fsend_sems.at[c],
                recv_sem=yrecv_sems.at[c],
                device_id=y_nbr,
                device_id_type=pl.DeviceIdType.MESH,
            )

        def consume(c):
            fwd(c).wait_recv()
            b, j = divmod(c, NC // B)
            rows = pl.ds(j * R, R)

            @pl.when(my_y == 0)
            def _():
                out_ref[b, rows, :C] = (
                    own[c % 2, :, :C].astype(jnp.float32)
                    + xrecv[c].astype(jnp.float32)
                ).astype(jnp.bfloat16)
                out_ref[b, rows, C:] = (
                    own[c % 2, :, C:].astype(jnp.float32)
                    + yrecv[c].astype(jnp.float32)
                ).astype(jnp.bfloat16)

            @pl.when(my_y == 1)
            def _():
                out_ref[b, rows, :C] = (
                    own[c % 2, :, :C].astype(jnp.float32)
                    + yrecv[c].astype(jnp.float32)
                ).astype(jnp.bfloat16)
                out_ref[b, rows, C:] = (
                    own[c % 2, :, C:].astype(jnp.float32)
                    + xrecv[c].astype(jnp.float32)
                ).astype(jnp.bfloat16)

        def head_gather(c, half_rows, land, sem):
            b, j = divmod(c, NC // B)
            copies = [
                pltpu.make_async_copy(
                    o_hbm.at[b, pl.ds(half_rows + j * R, R), h, :],
                    land.at[:, pl.ds(h * D, D)],
                    sem,
                )
                for h in range(H)
            ]
            for cp in copies:
                cp.start()
            return copies

        for c in range(NC):
            rem_copies = head_gather(c, rem_rows, land_rem, rem_sem)
            own_copies = head_gather(c, own_rows, land_own, own_sem)

            if c >= 2:
                consume(c - 2)

            if c >= 2:
                xrdma(c - 2).wait_send()

            for cp in rem_copies:
                cp.wait()
            o_rem = land_rem[...].astype(jnp.bfloat16)

            @pl.when(my_y == 0)
            def _():
                xsend[c % 2] = jnp.dot(
                    o_rem, wo_ref[:, :C],
                    preferred_element_type=jnp.float32,
                ).astype(jnp.bfloat16)

            @pl.when(my_y == 1)
            def _():
                xsend[c % 2] = jnp.dot(
                    o_rem, wo_ref[:, C:],
                    preferred_element_type=jnp.float32,
                ).astype(jnp.bfloat16)

            xrdma(c).start()

            for cp in own_copies:
                cp.wait()
            o_own = land_own[...].astype(jnp.bfloat16)
            own[c % 2, :, :C] = jnp.dot(
                o_own, wo_ref[:, :C], preferred_element_type=jnp.float32
            ).astype(jnp.bfloat16)
            own[c % 2, :, C:] = jnp.dot(
                o_own, wo_ref[:, C:], preferred_element_type=jnp.float32
            ).astype(jnp.bfloat16)

            xrdma(c).wait_recv()
            fwd(c).start()

        consume(NC - 2)
        consume(NC - 1)
        for c in range(NC - 2, NC):
            xrdma(c).wait_send()
        for c in range(NC):
            fwd(c).wait_send()

    return pl.pallas_call(
        body,
        out_shape=jax.ShapeDtypeStruct((B, S_HALF, HD_OUT), jnp.bfloat16),
        in_specs=[
            pl.BlockSpec(memory_space=pl.ANY),
            pl.BlockSpec(memory_space=pltpu.VMEM),
        ],
        out_specs=pl.BlockSpec(memory_space=pltpu.VMEM),
        scratch_shapes=[
            pltpu.VMEM((R, HD_IN), jnp.float32),
            pltpu.VMEM((R, HD_IN), jnp.float32),
            pltpu.VMEM((2, R, C), jnp.bfloat16),
            pltpu.VMEM((NC, R, C), jnp.bfloat16),
            pltpu.VMEM((NC, R, C), jnp.bfloat16),
            pltpu.VMEM((2, R, HD_OUT), jnp.bfloat16),
            pltpu.SemaphoreType.DMA((2,)),
            pltpu.SemaphoreType.DMA((NC,)),
            pltpu.SemaphoreType.DMA((NC,)),
            pltpu.SemaphoreType.DMA((NC,)),
            pltpu.SemaphoreType.DMA,
            pltpu.SemaphoreType.DMA,
        ],
        compiler_params=pltpu.CompilerParams(
            collective_id=0,
            vmem_limit_bytes=64 * 1024 * 1024,
        ),
    )(O, Wo2)


# device time: 195958 ns/iter; 1.0098x vs baseline; 1.0098x over previous
import jax
import jax.numpy as jnp
from jax import lax
from jax.experimental import pallas as pl
from jax.experimental.pallas import tpu as pltpu

B, S, HD_IN, HD_OUT = 4, 1024, 2048, 4096
S_HALF = S // 2
C = HD_OUT // 2
Q = 1024
NC = 8
R = B * S_HALF // NC


def kernel(O, Wo):
    H, D = O.shape[2], O.shape[3]
    Wo2 = Wo.astype(jnp.bfloat16)

    def body(
        o_hbm, wo_ref, out_hbm,
        land_rem, land_own, xsend, xrecv, yrecv, ysend, ystr, own, stage,
        xsend_sems, xrecv_sems, fsend_sems, yrecv_sems,
        ysend_sems, ystr_sems, rem_sem, own_sem, store_sem,
    ):
        my_x = lax.axis_index("x")
        my_y = lax.axis_index("y")
        x_nbr = (1 - my_x, my_y)
        y_nbr = (my_x, 1 - my_y)

        barrier_sem = pltpu.get_barrier_semaphore()
        for nbr in (x_nbr, y_nbr):
            pl.semaphore_signal(
                barrier_sem, inc=1,
                device_id=nbr, device_id_type=pl.DeviceIdType.MESH,
            )
        pl.semaphore_wait(barrier_sem, 2)

        own_rows = my_x * S_HALF
        rem_rows = (1 - my_x) * S_HALF

        def xrdma(c):
            return pltpu.make_async_remote_copy(
                src_ref=xsend.at[c % 2],
                dst_ref=xrecv.at[c],
                send_sem=xsend_sems.at[c % 2],
                recv_sem=xrecv_sems.at[c],
                device_id=x_nbr,
                device_id_type=pl.DeviceIdType.MESH,
            )

        def fwd(c):
            return pltpu.make_async_remote_copy(
                src_ref=xrecv.at[c],
                dst_ref=yrecv.at[c],
                send_sem=fsend_sems.at[c],
                recv_sem=yrecv_sems.at[c],
                device_id=y_nbr,
                device_id_type=pl.DeviceIdType.MESH,
            )

        def ystripe(c):
            return pltpu.make_async_remote_copy(
                src_ref=ysend.at[c % 2],
                dst_ref=ystr.at[c],
                send_sem=ysend_sems.at[c % 2],
                recv_sem=ystr_sems.at[c],
                device_id=y_nbr,
                device_id_type=pl.DeviceIdType.MESH,
            )

        def store(c):
            b, j = divmod(c, NC // B)
            return pltpu.make_async_copy(
                stage, out_hbm.at[b, pl.ds(j * R, R), :], store_sem
            )

        def consume(c):
            fwd(c).wait_recv()
            ystripe(c).wait_recv()
            if c > 0:
                store(c - 1).wait()

            f32 = jnp.float32

            @pl.when(my_y == 0)
            def _():
                stage[:, :C] = (
                    own[c % 2, :, :C].astype(f32) + xrecv[c].astype(f32)
                ).astype(jnp.bfloat16)
                stage[:, C:C + Q] = (
                    own[c % 2, :, C:C + Q].astype(f32)
                    + yrecv[c][:, :Q].astype(f32)
                ).astype(jnp.bfloat16)
                stage[:, C + Q:] = (
                    ystr[c].astype(f32) + yrecv[c][:, Q:].astype(f32)
                ).astype(jnp.bfloat16)

            @pl.when(my_y == 1)
            def _():
                stage[:, :Q] = (
                    ystr[c].astype(f32) + yrecv[c][:, :Q].astype(f32)
                ).astype(jnp.bfloat16)
                stage[:, Q:C] = (
                    own[c % 2, :, Q:C].astype(f32)
                    + yrecv[c][:, Q:].astype(f32)
                ).astype(jnp.bfloat16)
                stage[:, C:] = (
                    own[c % 2, :, C:].astype(f32) + xrecv[c].astype(f32)
                ).astype(jnp.bfloat16)

            store(c).start()

        def head_gather(c, half_rows, land, sem):
            b, j = divmod(c, NC // B)
            copies = [
                pltpu.make_async_copy(
                    o_hbm.at[b, pl.ds(half_rows + j * R, R), h, :],
                    land.at[:, pl.ds(h * D, D)],
                    sem,
                )
                for h in range(H)
            ]
            for cp in copies:
                cp.start()
            return copies

        for c in range(NC):
            rem_copies = head_gather(c, rem_rows, land_rem, rem_sem)
            own_copies = head_gather(c, own_rows, land_own, own_sem)

            if c >= 2:
                consume(c - 2)

            if c >= 2:
                xrdma(c - 2).wait_send()

            for cp in rem_copies:
                cp.wait()
            o_rem = land_rem[...].astype(jnp.bfloat16)

            @pl.when(my_y == 0)
            def _():
                xsend[c % 2] = jnp.dot(
                    o_rem, wo_ref[:, :C],
                    preferred_element_type=jnp.float32,
                ).astype(jnp.bfloat16)

            @pl.when(my_y == 1)
            def _():
                xsend[c % 2] = jnp.dot(
                    o_rem, wo_ref[:, C:],
                    preferred_element_type=jnp.float32,
                ).astype(jnp.bfloat16)

            xrdma(c).start()

            for cp in own_copies:
                cp.wait()
            o_own = land_own[...].astype(jnp.bfloat16)
            if c >= 2:
                ystripe(c - 2).wait_send()

            @pl.when(my_y == 0)
            def _():
                own[c % 2, :, :C] = jnp.dot(
                    o_own, wo_ref[:, :C], preferred_element_type=jnp.float32
                ).astype(jnp.bfloat16)
                own[c % 2, :, C:C + Q] = jnp.dot(
                    o_own, wo_ref[:, C:C + Q],
                    preferred_element_type=jnp.float32,
                ).astype(jnp.bfloat16)
                ysend[c % 2] = own[c % 2, :, :Q]

            @pl.when(my_y == 1)
            def _():
                own[c % 2, :, C:] = jnp.dot(
                    o_own, wo_ref[:, C:], preferred_element_type=jnp.float32
                ).astype(jnp.bfloat16)
                own[c % 2, :, Q:C] = jnp.dot(
                    o_own, wo_ref[:, Q:C],
                    preferred_element_type=jnp.float32,
                ).astype(jnp.bfloat16)
                ysend[c % 2] = own[c % 2, :, C + Q:]

            ystripe(c).start()

            xrdma(c).wait_recv()
            fwd(c).start()

        consume(NC - 2)
        consume(NC - 1)
        for c in range(NC - 2, NC):
            xrdma(c).wait_send()
            ystripe(c).wait_send()
        for c in range(NC):
            fwd(c).wait_send()
        store(NC - 1).wait()

    return pl.pallas_call(
        body,
        out_shape=jax.ShapeDtypeStruct((B, S_HALF, HD_OUT), jnp.bfloat16),
        in_specs=[
            pl.BlockSpec(memory_space=pl.ANY),
            pl.BlockSpec(memory_space=pltpu.VMEM),
        ],
        out_specs=pl.BlockSpec(memory_space=pl.ANY),
        scratch_shapes=[
            pltpu.VMEM((R, HD_IN), jnp.float32),
            pltpu.VMEM((R, HD_IN), jnp.float32),
            pltpu.VMEM((2, R, C), jnp.bfloat16),
            pltpu.VMEM((NC, R, C), jnp.bfloat16),
            pltpu.VMEM((NC, R, C), jnp.bfloat16),
            pltpu.VMEM((2, R, Q), jnp.bfloat16),
            pltpu.VMEM((NC, R, Q), jnp.bfloat16),
            pltpu.VMEM((2, R, HD_OUT), jnp.bfloat16),
            pltpu.VMEM((R, HD_OUT), jnp.bfloat16),
            pltpu.SemaphoreType.DMA((2,)),
            pltpu.SemaphoreType.DMA((NC,)),
            pltpu.SemaphoreType.DMA((NC,)),
            pltpu.SemaphoreType.DMA((NC,)),
            pltpu.SemaphoreType.DMA((2,)),
            pltpu.SemaphoreType.DMA((NC,)),
            pltpu.SemaphoreType.DMA,
            pltpu.SemaphoreType.DMA,
            pltpu.SemaphoreType.DMA,
        ],
        compiler_params=pltpu.CompilerParams(
            collective_id=0,
            vmem_limit_bytes=64 * 1024 * 1024,
        ),
    )(O, Wo2)


# device time: 195616 ns/iter; 1.0116x vs baseline; 1.0017x over previous
import jax
import jax.numpy as jnp
from jax import lax
from jax.experimental import pallas as pl
from jax.experimental.pallas import tpu as pltpu

B, S, HD_IN, HD_OUT = 4, 1024, 2048, 4096
S_HALF = S // 2
C = HD_OUT // 2
Q = 512
NC = 8
R = B * S_HALF // NC


def kernel(O, Wo):
    H, D = O.shape[2], O.shape[3]
    Wo2 = Wo.astype(jnp.bfloat16)

    def body(
        o_hbm, wo_ref, out_hbm,
        land_rem, land_own, xsend, xrecv, yrecv, ysend, ystr, own, stage,
        xsend_sems, xrecv_sems, fsend_sems, yrecv_sems,
        ysend_sems, ystr_sems, rem_sem, own_sem, store_sem,
    ):
        my_x = lax.axis_index("x")
        my_y = lax.axis_index("y")
        x_nbr = (1 - my_x, my_y)
        y_nbr = (my_x, 1 - my_y)

        barrier_sem = pltpu.get_barrier_semaphore()
        for nbr in (x_nbr, y_nbr):
            pl.semaphore_signal(
                barrier_sem, inc=1,
                device_id=nbr, device_id_type=pl.DeviceIdType.MESH,
            )
        pl.semaphore_wait(barrier_sem, 2)

        own_rows = my_x * S_HALF
        rem_rows = (1 - my_x) * S_HALF

        def xrdma(c):
            return pltpu.make_async_remote_copy(
                src_ref=xsend.at[c % 2],
                dst_ref=xrecv.at[c],
                send_sem=xsend_sems.at[c % 2],
                recv_sem=xrecv_sems.at[c],
                device_id=x_nbr,
                device_id_type=pl.DeviceIdType.MESH,
            )

        def fwd(c):
            return pltpu.make_async_remote_copy(
                src_ref=xrecv.at[c],
                dst_ref=yrecv.at[c],
                send_sem=fsend_sems.at[c],
                recv_sem=yrecv_sems.at[c],
                device_id=y_nbr,
                device_id_type=pl.DeviceIdType.MESH,
            )

        def ystripe(c):
            return pltpu.make_async_remote_copy(
                src_ref=ysend.at[c % 2],
                dst_ref=ystr.at[c],
                send_sem=ysend_sems.at[c % 2],
                recv_sem=ystr_sems.at[c],
                device_id=y_nbr,
                device_id_type=pl.DeviceIdType.MESH,
            )

        def store(c):
            b, j = divmod(c, NC // B)
            return pltpu.make_async_copy(
                stage, out_hbm.at[b, pl.ds(j * R, R), :], store_sem
            )

        def consume(c):
            fwd(c).wait_recv()
            ystripe(c).wait_recv()
            if c > 0:
                store(c - 1).wait()

            f32 = jnp.float32

            @pl.when(my_y == 0)
            def _():
                stage[:, :C] = (
                    own[c % 2, :, :C].astype(f32) + xrecv[c].astype(f32)
                ).astype(jnp.bfloat16)
                stage[:, C:HD_OUT - Q] = (
                    own[c % 2, :, C:HD_OUT - Q].astype(f32)
                    + yrecv[c][:, :C - Q].astype(f32)
                ).astype(jnp.bfloat16)
                stage[:, HD_OUT - Q:] = (
                    ystr[c].astype(f32) + yrecv[c][:, C - Q:].astype(f32)
                ).astype(jnp.bfloat16)

            @pl.when(my_y == 1)
            def _():
                stage[:, :Q] = (
                    ystr[c].astype(f32) + yrecv[c][:, :Q].astype(f32)
                ).astype(jnp.bfloat16)
                stage[:, Q:C] = (
                    own[c % 2, :, Q:C].astype(f32)
                    + yrecv[c][:, Q:].astype(f32)
                ).astype(jnp.bfloat16)
                stage[:, C:] = (
                    own[c % 2, :, C:].astype(f32) + xrecv[c].astype(f32)
                ).astype(jnp.bfloat16)

            store(c).start()

        def head_gather(c, half_rows, land, sem):
            b, j = divmod(c, NC // B)
            copies = [
                pltpu.make_async_copy(
                    o_hbm.at[b, pl.ds(half_rows + j * R, R), h, :],
                    land.at[:, pl.ds(h * D, D)],
                    sem,
                )
                for h in range(H)
            ]
            for cp in copies:
                cp.start()
            return copies

        for c in range(NC):
            rem_copies = head_gather(c, rem_rows, land_rem, rem_sem)
            own_copies = head_gather(c, own_rows, land_own, own_sem)

            if c >= 2:
                consume(c - 2)

            if c >= 2:
                xrdma(c - 2).wait_send()

            for cp in rem_copies:
                cp.wait()
            o_rem = land_rem[...].astype(jnp.bfloat16)

            @pl.when(my_y == 0)
            def _():
                xsend[c % 2] = jnp.dot(
                    o_rem, wo_ref[:, :C],
                    preferred_element_type=jnp.float32,
                ).astype(jnp.bfloat16)

            @pl.when(my_y == 1)
            def _():
                xsend[c % 2] = jnp.dot(
                    o_rem, wo_ref[:, C:],
                    preferred_element_type=jnp.float32,
                ).astype(jnp.bfloat16)

            xrdma(c).start()

            for cp in own_copies:
                cp.wait()
            o_own = land_own[...].astype(jnp.bfloat16)
            if c >= 2:
                ystripe(c - 2).wait_send()

            @pl.when(my_y == 0)
            def _():
                own[c % 2, :, :C] = jnp.dot(
                    o_own, wo_ref[:, :C], preferred_element_type=jnp.float32
                ).astype(jnp.bfloat16)
                own[c % 2, :, C:HD_OUT - Q] = jnp.dot(
                    o_own, wo_ref[:, C:HD_OUT - Q],
                    preferred_element_type=jnp.float32,
                ).astype(jnp.bfloat16)
                ysend[c % 2] = own[c % 2, :, :Q]

            @pl.when(my_y == 1)
            def _():
                own[c % 2, :, C:] = jnp.dot(
                    o_own, wo_ref[:, C:], preferred_element_type=jnp.float32
                ).astype(jnp.bfloat16)
                own[c % 2, :, Q:C] = jnp.dot(
                    o_own, wo_ref[:, Q:C],
                    preferred_element_type=jnp.float32,
                ).astype(jnp.bfloat16)
                ysend[c % 2] = own[c % 2, :, HD_OUT - Q:]

            ystripe(c).start()

            xrdma(c).wait_recv()
            fwd(c).start()

        consume(NC - 2)
        consume(NC - 1)
        for c in range(NC - 2, NC):
            xrdma(c).wait_send()
            ystripe(c).wait_send()
        for c in range(NC):
            fwd(c).wait_send()
        store(NC - 1).wait()

    return pl.pallas_call(
        body,
        out_shape=jax.ShapeDtypeStruct((B, S_HALF, HD_OUT), jnp.bfloat16),
        in_specs=[
            pl.BlockSpec(memory_space=pl.ANY),
            pl.BlockSpec(memory_space=pltpu.VMEM),
        ],
        out_specs=pl.BlockSpec(memory_space=pl.ANY),
        scratch_shapes=[
            pltpu.VMEM((R, HD_IN), jnp.float32),
            pltpu.VMEM((R, HD_IN), jnp.float32),
            pltpu.VMEM((2, R, C), jnp.bfloat16),
            pltpu.VMEM((NC, R, C), jnp.bfloat16),
            pltpu.VMEM((NC, R, C), jnp.bfloat16),
            pltpu.VMEM((2, R, Q), jnp.bfloat16),
            pltpu.VMEM((NC, R, Q), jnp.bfloat16),
            pltpu.VMEM((2, R, HD_OUT), jnp.bfloat16),
            pltpu.VMEM((R, HD_OUT), jnp.bfloat16),
            pltpu.SemaphoreType.DMA((2,)),
            pltpu.SemaphoreType.DMA((NC,)),
            pltpu.SemaphoreType.DMA((NC,)),
            pltpu.SemaphoreType.DMA((NC,)),
            pltpu.SemaphoreType.DMA((2,)),
            pltpu.SemaphoreType.DMA((NC,)),
            pltpu.SemaphoreType.DMA,
            pltpu.SemaphoreType.DMA,
            pltpu.SemaphoreType.DMA,
        ],
        compiler_params=pltpu.CompilerParams(
            collective_id=0,
            vmem_limit_bytes=64 * 1024 * 1024,
        ),
    )(O, Wo2)


# device time: 194719 ns/iter; 1.0163x vs baseline; 1.0046x over previous
import jax
import jax.numpy as jnp
from jax import lax
from jax.experimental import pallas as pl
from jax.experimental.pallas import tpu as pltpu

B, S, HD_IN, HD_OUT = 4, 1024, 2048, 4096
S_HALF = S // 2
C = HD_OUT // 2


def kernel(O, Wo):
    H, D = O.shape[2], O.shape[3]
    Wo2 = Wo.astype(jnp.bfloat16)

    def body(
        o_hbm, wo_ref, out_hbm,
        land_rem, land_own, xsend, xrecv, yrecv, own, stage,
        xsend_sems, xrecv_sems, fsend_sems, yrecv_sems,
        rem_sem, own_sem, store_sem,
    ):
        my_x = lax.axis_index("x")
        my_y = lax.axis_index("y")
        x_nbr = (1 - my_x, my_y)
        y_nbr = (my_x, 1 - my_y)

        barrier_sem = pltpu.get_barrier_semaphore()
        for nbr in (x_nbr, y_nbr):
            pl.semaphore_signal(
                barrier_sem, inc=1,
                device_id=nbr, device_id_type=pl.DeviceIdType.MESH,
            )
        pl.semaphore_wait(barrier_sem, 2)

        own_rows = my_x * S_HALF
        rem_rows = (1 - my_x) * S_HALF

        def xrdma(b):
            return pltpu.make_async_remote_copy(
                src_ref=xsend.at[b % 2],
                dst_ref=xrecv.at[b],
                send_sem=xsend_sems.at[b % 2],
                recv_sem=xrecv_sems.at[b],
                device_id=x_nbr,
                device_id_type=pl.DeviceIdType.MESH,
            )

        def fwd(b):
            return pltpu.make_async_remote_copy(
                src_ref=xrecv.at[b],
                dst_ref=yrecv.at[b],
                send_sem=fsend_sems.at[b],
                recv_sem=yrecv_sems.at[b],
                device_id=y_nbr,
                device_id_type=pl.DeviceIdType.MESH,
            )

        def store(b):
            return pltpu.make_async_copy(stage, out_hbm.at[b], store_sem)

        def consume(b):
            fwd(b).wait_recv()
            if b > 0:
                store(b - 1).wait()

            @pl.when(my_y == 0)
            def _():
                stage[:, :C] = (
                    own[b % 2, :, :C].astype(jnp.float32)
                    + xrecv[b].astype(jnp.float32)
                ).astype(jnp.bfloat16)
                stage[:, C:] = (
                    own[b % 2, :, C:].astype(jnp.float32)
                    + yrecv[b].astype(jnp.float32)
                ).astype(jnp.bfloat16)

            @pl.when(my_y == 1)
            def _():
                stage[:, :C] = (
                    own[b % 2, :, :C].astype(jnp.float32)
                    + yrecv[b].astype(jnp.float32)
                ).astype(jnp.bfloat16)
                stage[:, C:] = (
                    own[b % 2, :, C:].astype(jnp.float32)
                    + xrecv[b].astype(jnp.float32)
                ).astype(jnp.bfloat16)

            store(b).start()

        def head_gather(b, rows, land, sem):
            copies = [
                pltpu.make_async_copy(
                    o_hbm.at[b, pl.ds(rows, S_HALF), h, :],
                    land.at[:, pl.ds(h * D, D)],
                    sem,
                )
                for h in range(H)
            ]
            for c in copies:
                c.start()
            return copies

        for b in range(B):
            rem_copies = head_gather(b, rem_rows, land_rem, rem_sem)
            own_copies = head_gather(b, own_rows, land_own, own_sem)

            if b >= 2:
                consume(b - 2)

            if b >= 2:
                xrdma(b - 2).wait_send()

            for c in rem_copies:
                c.wait()
            o_rem = land_rem[...].astype(jnp.bfloat16)

            @pl.when(my_y == 0)
            def _():
                xsend[b % 2] = jnp.dot(
                    o_rem, wo_ref[:, :C],
                    preferred_element_type=jnp.float32,
                ).astype(jnp.bfloat16)

            @pl.when(my_y == 1)
            def _():
                xsend[b % 2] = jnp.dot(
                    o_rem, wo_ref[:, C:],
                    preferred_element_type=jnp.float32,
                ).astype(jnp.bfloat16)

            xrdma(b).start()

            for c in own_copies:
                c.wait()
            o_own = land_own[...].astype(jnp.bfloat16)
            own[b % 2, :, :C] = jnp.dot(
                o_own, wo_ref[:, :C], preferred_element_type=jnp.float32
            ).astype(jnp.bfloat16)
            own[b % 2, :, C:] = jnp.dot(
                o_own, wo_ref[:, C:], preferred_element_type=jnp.float32
            ).astype(jnp.bfloat16)

            xrdma(b).wait_recv()
            fwd(b).start()

        consume(B - 2)
        consume(B - 1)
        for b in range(B - 2, B):
            xrdma(b).wait_send()
        for b in range(B):
            fwd(b).wait_send()
        store(B - 1).wait()

    return pl.pallas_call(
        body,
        out_shape=jax.ShapeDtypeStruct((B, S_HALF, HD_OUT), jnp.bfloat16),
        in_specs=[
            pl.BlockSpec(memory_space=pl.ANY),
            pl.BlockSpec(memory_space=pltpu.VMEM),
        ],
        out_specs=pl.BlockSpec(memory_space=pl.ANY),
        scratch_shapes=[
            pltpu.VMEM((S_HALF, HD_IN), jnp.float32),
            pltpu.VMEM((S_HALF, HD_IN), jnp.float32),
            pltpu.VMEM((2, S_HALF, C), jnp.bfloat16),
            pltpu.VMEM((B, S_HALF, C), jnp.bfloat16),
            pltpu.VMEM((B, S_HALF, C), jnp.bfloat16),
            pltpu.VMEM((2, S_HALF, HD_OUT), jnp.bfloat16),
            pltpu.VMEM((S_HALF, HD_OUT), jnp.bfloat16),
            pltpu.SemaphoreType.DMA((2,)),
            pltpu.SemaphoreType.DMA((B,)),
            pltpu.SemaphoreType.DMA((B,)),
            pltpu.SemaphoreType.DMA((B,)),
            pltpu.SemaphoreType.DMA,
            pltpu.SemaphoreType.DMA,
            pltpu.SemaphoreType.DMA,
        ],
        compiler_params=pltpu.CompilerParams(
            collective_id=0,
            vmem_limit_bytes=64 * 1024 * 1024,
        ),
    )(O, Wo2)
